# baseline (speedup 1.0000x reference)
import functools

import jax
import jax.numpy as jnp
from jax import lax
from jax.experimental import pallas as pl
from jax.experimental.pallas import tpu as pltpu
from jax.experimental.pallas import tpu_sc as plsc

N = 1_000_000
D = 64
NC = 2
NS = 16
NW = NC * NS
L = 16

CHUNK = 400
MAIN_PER_TILE = 78
ROWS_PER_TILE = CHUNK * MAIN_PER_TILE
MAIN_ROWS = ROWS_PER_TILE * NW
TAIL_CHUNKS = (N - MAIN_ROWS) // CHUNK
GROUPS = CHUNK // L


def _rsqrt16(x):
    xi = plsc.bitcast(x, jnp.int32)
    y = plsc.bitcast(jnp.int32(0x5F3759DF) - (xi >> 1), jnp.float32)
    xh = x * jnp.float32(-0.5)
    for _ in range(3):
        y = y * (jnp.float32(1.5) + xh * y * y)
    return y


def _sc_body(cand_hbm, slots_hbm, scores_hbm, pmax_hbm, pidx_hbm,
             cand_v, in_v, sc_v, dbuf, nbuf, mvec, ivec):
    c = lax.axis_index("c")
    s = lax.axis_index("s")
    wid = s * NC + c

    pltpu.sync_copy(cand_hbm, cand_v)
    c0 = cand_v[pl.ds(0, L)]
    c1 = cand_v[pl.ds(L, L)]
    c2 = cand_v[pl.ds(2 * L, L)]
    c3 = cand_v[pl.ds(3 * L, L)]
    cn2 = jnp.sum(c0 * c0 + c1 * c1 + c2 * c2 + c3 * c3)
    inv_c = _rsqrt16(jnp.full((L,), jnp.maximum(cn2, jnp.float32(1e-30)),
                              jnp.float32))
    c0 = c0 * inv_c
    c1 = c1 * inv_c
    c2 = c2 * inv_c
    c3 = c3 * inv_c

    mvec[...] = jnp.full((L,), -jnp.inf, jnp.float32)
    ivec[...] = jnp.zeros((L,), jnp.int32)
    iota = lax.iota(jnp.int32, L)
    gidx = iota * L + (L - 1)

    def process_chunk(row0):
        pltpu.sync_copy(slots_hbm.at[pl.ds(row0, CHUNK), :], in_v)

        def group(g, carry):
            for r in range(L):
                i = g * L + r
                v0 = in_v[i, pl.ds(0, L)]
                v1 = in_v[i, pl.ds(L, L)]
                v2 = in_v[i, pl.ds(2 * L, L)]
                v3 = in_v[i, pl.ds(3 * L, L)]
                sv = v0 * c0 + v1 * c1 + v2 * c2 + v3 * c3
                nv = v0 * v0 + v1 * v1 + v2 * v2 + v3 * v3
                dbuf[pl.ds(r * L, L)] = jnp.cumsum(sv)
                nbuf[pl.ds(r * L, L)] = jnp.cumsum(nv)
            dvec = plsc.load_gather(dbuf, [gidx])
            nvec = plsc.load_gather(nbuf, [gidx])
            sc16 = dvec * _rsqrt16(jnp.maximum(nvec, jnp.float32(1e-30)))
            sc_v[pl.ds(g * L, L)] = sc16
            idx16 = iota + (row0 + g * L)
            m = mvec[...]
            better = sc16 > m
            mvec[...] = jnp.where(better, sc16, m)
            ivec[...] = jnp.where(better, idx16, ivec[...])
            return carry

        lax.fori_loop(0, GROUPS, group, 0)
        pltpu.sync_copy(sc_v, scores_hbm.at[pl.ds(row0, CHUNK)])

    base = wid * ROWS_PER_TILE

    def chunk_loop(k, carry):
        process_chunk(base + k * CHUNK)
        return carry

    lax.fori_loop(0, MAIN_PER_TILE, chunk_loop, 0)

    @pl.when(wid < TAIL_CHUNKS)
    def _():
        process_chunk(MAIN_ROWS + wid * CHUNK)

    pltpu.sync_copy(mvec, pmax_hbm.at[wid])
    pltpu.sync_copy(ivec, pidx_hbm.at[wid])


def _merge_body(pm_ref, pi_ref, o_ref):
    m = pm_ref[...]
    i = pi_ref[...]
    best = jnp.max(m)
    o_ref[0, 0] = jnp.min(jnp.where(m == best, i, jnp.int32(2147483647)))


def _merge(pmax, pidx):
    return pl.pallas_call(
        _merge_body,
        out_shape=jax.ShapeDtypeStruct((1, 1), jnp.int32),
        out_specs=pl.BlockSpec(memory_space=pltpu.SMEM),
    )(pmax, pidx)


@jax.jit
def kernel(candidate, slot_embeds):
    mesh = plsc.VectorSubcoreMesh(core_axis_name="c", subcore_axis_name="s")
    sc_call = pl.kernel(
        _sc_body,
        out_type=[
            jax.ShapeDtypeStruct((N,), jnp.float32),
            jax.ShapeDtypeStruct((NW, L), jnp.float32),
            jax.ShapeDtypeStruct((NW, L), jnp.int32),
        ],
        scratch_types=[
            pltpu.VMEM((D,), jnp.float32),
            pltpu.VMEM((CHUNK, D), jnp.float32),
            pltpu.VMEM((CHUNK,), jnp.float32),
            pltpu.VMEM((L * L,), jnp.float32),
            pltpu.VMEM((L * L,), jnp.float32),
            pltpu.VMEM((L,), jnp.float32),
            pltpu.VMEM((L,), jnp.int32),
        ],
        mesh=mesh,
        compiler_params=pltpu.CompilerParams(needs_layout_passes=False),
    )
    scores, pmax, pidx = sc_call(candidate, slot_embeds)
    best = _merge(pmax, pidx)[0, 0]
    return scores, best
